# baseline (speedup 1.0000x reference)
import functools
import math

import jax
import jax.numpy as jnp
import numpy as np
from jax.experimental import pallas as pl
from jax.experimental.pallas import tpu as pltpu

_B, _N, _C = 8, 4096, 64
_S, _K = 1024, 32
_IN_DIM = 3
_OUT_DIM = 128
_SIGMA = 0.26
_BASELINE = 0.1
_SCALING = 10.0
_EPS = 1e-06

_feat_dim = math.ceil(_OUT_DIM / _IN_DIM)
_feat_num = _feat_dim * _IN_DIM
_OUT_IDX_NP = np.linspace(0, _feat_num - 1, _OUT_DIM).astype(np.int64)
_FEAT_VAL_NP = np.linspace(-1.0, 1.0, _feat_dim + 2)[1:-1].astype(np.float32)
_COORD_OF_LANE = (_OUT_IDX_NP // _feat_dim).astype(np.int32)
_FVV_NP = _FEAT_VAL_NP[(_OUT_IDX_NP % _feat_dim)].astype(np.float32)
assert (_COORD_OF_LANE == np.where(np.arange(128) < 43, 0,
                                   np.where(np.arange(128) < 86, 1, 2))).all()

_NX = _B * _S * _IN_DIM
_NF = _B * _S * _C
_SK = _S * _K


_SD = 32


def _moments_body(xk_ref, xs_ref, fk_ref, fs_ref, xmom_ref, fmom_ref):
    b = pl.program_id(0)
    st = pl.program_id(1)
    cx = xk_ref[...] - xs_ref[...]
    cf = fk_ref[...] - fs_ref[...]
    xs0 = jnp.sum(cx, axis=0)
    xs1 = jnp.sum(cx * cx, axis=0)
    xm = jnp.stack([xs0, xs1], axis=0)[None]
    fs0 = jnp.sum(cf, axis=(0, 2))
    fs1 = jnp.sum(cf * cf, axis=(0, 2))
    fm = jnp.stack([fs0, fs1], axis=0)

    @pl.when(st == 0)
    def _():
        xmom_ref[...] = xm

    @pl.when(st != 0)
    def _():
        xmom_ref[...] += xm

    @pl.when(jnp.logical_and(b == 0, st == 0))
    def _():
        fmom_ref[...] = fm

    @pl.when(jnp.logical_not(jnp.logical_and(b == 0, st == 0)))
    def _():
        fmom_ref[...] += fm


def _compute_moments(xk, xs, fk, fs):
    grid = (_B, _S // _SD)
    return pl.pallas_call(
        _moments_body,
        grid=grid,
        in_specs=[
            pl.BlockSpec((_SD, _K, _IN_DIM), lambda b, st: (b * (_S // _SD) + st, 0, 0)),
            pl.BlockSpec((_SD, 1, _IN_DIM), lambda b, st: (b * (_S // _SD) + st, 0, 0)),
            pl.BlockSpec((_SD, _K, _C), lambda b, st: (b * (_S // _SD) + st, 0, 0)),
            pl.BlockSpec((_SD, 1, _C), lambda b, st: (b * (_S // _SD) + st, 0, 0)),
        ],
        out_specs=[
            pl.BlockSpec((1, 2, _K, _IN_DIM), lambda b, st: (b, 0, 0, 0)),
            pl.BlockSpec((2, _K), lambda b, st: (0, 0)),
        ],
        out_shape=[
            jax.ShapeDtypeStruct((_B, 2, _K, _IN_DIM), jnp.float32),
            jax.ShapeDtypeStruct((2, _K), jnp.float32),
        ],
    )(xk, xs, fk, fs)


def _finalize_body(xmom_ref, fmom_ref, out_ref):
    xm = xmom_ref[...]
    fm = fmom_ref[...]
    ts = jnp.sum(xm[:, 0], axis=(0, 2))
    tq = jnp.sum(xm[:, 1], axis=(0, 2))
    nx = jnp.float32(_NX)
    varx = (tq - ts * ts / nx) / (nx - 1.0)
    stdx = jnp.maximum(jnp.sqrt(jnp.maximum(varx, 0.0)), 1e-5)
    invx = 1.0 / stdx

    nf = jnp.float32(_NF)
    varf = (fm[1] - fm[0] * fm[0] / nf) / (nf - 1.0)
    stdf = jnp.maximum(jnp.sqrt(jnp.maximum(varf, 0.0)), 1e-5)
    invf = 1.0 / stdf

    sb = jnp.sum(xm[:, 0] * invx[None, :, None], axis=1)
    qb = jnp.sum(xm[:, 1] * (invx * invx)[None, :, None], axis=1)
    nsk = jnp.float32(_SK)
    var_bi = (qb - sb * sb / nsk) / (nsk - 1.0)
    std_bi = jnp.sqrt(jnp.maximum(var_bi, 0.0))
    gs = jnp.mean(std_bi)
    sigma = _SIGMA * (1.0 + gs)
    inv_sigma = 1.0 / (sigma + _EPS)
    blend = jax.nn.sigmoid((gs - _BASELINE) * _SCALING)

    ones = jnp.ones((_K,), jnp.float32)
    out_ref[...] = jnp.stack([invx, invf, inv_sigma * ones, blend * ones], axis=0)


def _finalize_stats(xmom, fmom):
    return pl.pallas_call(
        _finalize_body,
        out_shape=jax.ShapeDtypeStruct((4, _K), jnp.float32),
    )(xmom, fmom)


_SE = 8


def _tail_body(xk_ref, xs_ref, fk_ref, fs_ref, st_ref, fvv_ref, out_ref):
    st = st_ref[...]
    invx = st[0][None, :, None]
    invf = st[1][None, :, None]
    inv_sigma = st[2:3, 0:1].reshape(1, 1, 1)
    blend = st[3:4, 0:1].reshape(1, 1, 1)

    x = xk_ref[...]
    xn = (x - xs_ref[...]) * invx

    lane = jax.lax.broadcasted_iota(jnp.int32, (1, 1, _OUT_DIM), 2)
    x0 = xn[:, :, 0:1]
    x1 = xn[:, :, 1:2]
    x2 = xn[:, :, 2:3]
    sel = jnp.where(lane < 43, x0, jnp.where(lane < 86, x1, x2))
    t = (sel - fvv_ref[...]) * inv_sigma
    rbf = jnp.exp(-0.5 * (t * t))
    cosn = jnp.cos(t)
    pe = blend * rbf + (1.0 - blend) * cosn

    f = fk_ref[...]
    fs = fs_ref[...]
    fn = (f - fs) * invf
    fsb = jnp.broadcast_to(fs, (fn.shape[0], _K, _C))
    fcat = jnp.concatenate([fn, fsb], axis=-1)

    w = (fcat + pe) * pe
    agg = jnp.mean(w, axis=1) + jnp.max(w, axis=1)
    out_ref[...] = 0.5 * agg * (1.0 + jax.lax.erf(agg * np.float32(1.0 / np.sqrt(2.0))))


def _fused_tail(xk, xs, fk, fs, stats):
    grid = (_B * _S // _SE,)
    return pl.pallas_call(
        _tail_body,
        grid=grid,
        in_specs=[
            pl.BlockSpec((_SE, _K, _IN_DIM), lambda i: (i, 0, 0)),
            pl.BlockSpec((_SE, 1, _IN_DIM), lambda i: (i, 0, 0)),
            pl.BlockSpec((_SE, _K, _C), lambda i: (i, 0, 0)),
            pl.BlockSpec((_SE, 1, _C), lambda i: (i, 0, 0)),
            pl.BlockSpec((4, _K), lambda i: (0, 0)),
            pl.BlockSpec((1, 1, _OUT_DIM), lambda i: (0, 0, 0)),
        ],
        out_specs=pl.BlockSpec((_SE, _OUT_DIM), lambda i: (i, 0)),
        out_shape=jax.ShapeDtypeStruct((_B * _S, _OUT_DIM), jnp.float32),
    )(xk, xs, fk, fs, stats, jnp.asarray(_FVV_NP).reshape(1, 1, _OUT_DIM))


def _fps_jax(xyz):
    def one(x):
        def step(carry, _):
            dist, last = carry
            idx = last
            d = jnp.sum((x - x[last]) ** 2, axis=-1)
            dist = jnp.minimum(dist, d)
            nxt = jnp.argmax(dist).astype(jnp.int32)
            return (dist, nxt), idx
        (_, _), idxs = jax.lax.scan(
            step, (jnp.full((x.shape[0],), 1e10, x.dtype), jnp.int32(0)),
            None, length=_S)
        return idxs
    return jax.vmap(one)(xyz)


def _knn_jax(xyz, new_xyz):
    d = -2.0 * jnp.matmul(new_xyz, jnp.swapaxes(xyz, 1, 2))
    d = d + jnp.sum(new_xyz ** 2, axis=-1)[:, :, None]
    d = d + jnp.sum(xyz ** 2, axis=-1)[:, None, :]
    _, gi = jax.lax.top_k(-d, _K)
    return gi


def kernel(xyz, feat):
    fps_idx = _fps_jax(xyz)
    take = jax.vmap(lambda p, i: p[i])
    xyz_s = take(xyz, fps_idx)
    feat_s = take(feat, fps_idx)
    idx_knn = _knn_jax(xyz, xyz_s)
    xyz_k = take(xyz, idx_knn.reshape(_B, -1)).reshape(_B, _S, _K, _IN_DIM)
    feat_k = take(feat, idx_knn.reshape(_B, -1)).reshape(_B, _S, _K, _C)

    xk = xyz_k.reshape(_B * _S, _K, _IN_DIM)
    xs = xyz_s.reshape(_B * _S, 1, _IN_DIM)
    fk = feat_k.reshape(_B * _S, _K, _C)
    fs = feat_s.reshape(_B * _S, 1, _C)

    xmom, fmom = _compute_moments(xk, xs, fk, fs)
    stats = _finalize_stats(xmom, fmom)
    out = _fused_tail(xk, xs, fk, fs, stats)
    return out.reshape(_B, _S, _OUT_DIM)


# speedup vs baseline: 1.5211x; 1.5211x over previous
import functools
import math

import jax
import jax.numpy as jnp
import numpy as np
from jax.experimental import pallas as pl
from jax.experimental.pallas import tpu as pltpu

_B, _N, _C = 8, 4096, 64
_S, _K = 1024, 32
_IN_DIM = 3
_OUT_DIM = 128
_SIGMA = 0.26
_BASELINE = 0.1
_SCALING = 10.0
_EPS = 1e-06

_feat_dim = math.ceil(_OUT_DIM / _IN_DIM)
_feat_num = _feat_dim * _IN_DIM
_OUT_IDX_NP = np.linspace(0, _feat_num - 1, _OUT_DIM).astype(np.int64)
_FEAT_VAL_NP = np.linspace(-1.0, 1.0, _feat_dim + 2)[1:-1].astype(np.float32)
_COORD_OF_LANE = (_OUT_IDX_NP // _feat_dim).astype(np.int32)
_FVV_NP = _FEAT_VAL_NP[(_OUT_IDX_NP % _feat_dim)].astype(np.float32)
assert (_COORD_OF_LANE == np.where(np.arange(128) < 43, 0,
                                   np.where(np.arange(128) < 86, 1, 2))).all()

_NX = _B * _S * _IN_DIM
_NF = _B * _S * _C
_SK = _S * _K


_SD = 32


def _moments_body(xk_ref, xs_ref, fk_ref, fs_ref, xmom_ref, fmom_ref):
    b = pl.program_id(0)
    st = pl.program_id(1)
    cx = xk_ref[...] - xs_ref[...]
    cf = fk_ref[...] - fs_ref[...]
    xs0 = jnp.sum(cx, axis=0)
    xs1 = jnp.sum(cx * cx, axis=0)
    xm = jnp.stack([xs0, xs1], axis=0)[None]
    fs0 = jnp.sum(cf, axis=(0, 2))
    fs1 = jnp.sum(cf * cf, axis=(0, 2))
    fm = jnp.stack([fs0, fs1], axis=0)

    @pl.when(st == 0)
    def _():
        xmom_ref[...] = xm

    @pl.when(st != 0)
    def _():
        xmom_ref[...] += xm

    @pl.when(jnp.logical_and(b == 0, st == 0))
    def _():
        fmom_ref[...] = fm

    @pl.when(jnp.logical_not(jnp.logical_and(b == 0, st == 0)))
    def _():
        fmom_ref[...] += fm


def _compute_moments(xk, xs, fk, fs):
    grid = (_B, _S // _SD)
    return pl.pallas_call(
        _moments_body,
        grid=grid,
        in_specs=[
            pl.BlockSpec((_SD, _K, _IN_DIM), lambda b, st: (b * (_S // _SD) + st, 0, 0)),
            pl.BlockSpec((_SD, 1, _IN_DIM), lambda b, st: (b * (_S // _SD) + st, 0, 0)),
            pl.BlockSpec((_SD, _K, _C), lambda b, st: (b * (_S // _SD) + st, 0, 0)),
            pl.BlockSpec((_SD, 1, _C), lambda b, st: (b * (_S // _SD) + st, 0, 0)),
        ],
        out_specs=[
            pl.BlockSpec((1, 2, _K, _IN_DIM), lambda b, st: (b, 0, 0, 0)),
            pl.BlockSpec((2, _K), lambda b, st: (0, 0)),
        ],
        out_shape=[
            jax.ShapeDtypeStruct((_B, 2, _K, _IN_DIM), jnp.float32),
            jax.ShapeDtypeStruct((2, _K), jnp.float32),
        ],
    )(xk, xs, fk, fs)


def _finalize_body(xmom_ref, fmom_ref, out_ref):
    xm = xmom_ref[...]
    fm = fmom_ref[...]
    ts = jnp.sum(xm[:, 0], axis=(0, 2))
    tq = jnp.sum(xm[:, 1], axis=(0, 2))
    nx = jnp.float32(_NX)
    varx = (tq - ts * ts / nx) / (nx - 1.0)
    stdx = jnp.maximum(jnp.sqrt(jnp.maximum(varx, 0.0)), 1e-5)
    invx = 1.0 / stdx

    nf = jnp.float32(_NF)
    varf = (fm[1] - fm[0] * fm[0] / nf) / (nf - 1.0)
    stdf = jnp.maximum(jnp.sqrt(jnp.maximum(varf, 0.0)), 1e-5)
    invf = 1.0 / stdf

    sb = jnp.sum(xm[:, 0] * invx[None, :, None], axis=1)
    qb = jnp.sum(xm[:, 1] * (invx * invx)[None, :, None], axis=1)
    nsk = jnp.float32(_SK)
    var_bi = (qb - sb * sb / nsk) / (nsk - 1.0)
    std_bi = jnp.sqrt(jnp.maximum(var_bi, 0.0))
    gs = jnp.mean(std_bi)
    sigma = _SIGMA * (1.0 + gs)
    inv_sigma = 1.0 / (sigma + _EPS)
    blend = jax.nn.sigmoid((gs - _BASELINE) * _SCALING)

    ones = jnp.ones((_K,), jnp.float32)
    out_ref[...] = jnp.stack([invx, invf, inv_sigma * ones, blend * ones], axis=0)


def _finalize_stats(xmom, fmom):
    return pl.pallas_call(
        _finalize_body,
        out_shape=jax.ShapeDtypeStruct((4, _K), jnp.float32),
    )(xmom, fmom)


_SE = 8


def _tail_body(xk_ref, xs_ref, fk_ref, fs_ref, st_ref, fvv_ref, out_ref):
    st = st_ref[...]
    invx = st[0][None, :, None]
    invf = st[1][None, :, None]
    inv_sigma = st[2:3, 0:1].reshape(1, 1, 1)
    blend = st[3:4, 0:1].reshape(1, 1, 1)

    x = xk_ref[...]
    xn = (x - xs_ref[...]) * invx

    lane = jax.lax.broadcasted_iota(jnp.int32, (1, 1, _OUT_DIM), 2)
    x0 = xn[:, :, 0:1]
    x1 = xn[:, :, 1:2]
    x2 = xn[:, :, 2:3]
    sel = jnp.where(lane < 43, x0, jnp.where(lane < 86, x1, x2))
    t = (sel - fvv_ref[...]) * inv_sigma
    rbf = jnp.exp(-0.5 * (t * t))
    cosn = jnp.cos(t)
    pe = blend * rbf + (1.0 - blend) * cosn

    f = fk_ref[...]
    fs = fs_ref[...]
    fn = (f - fs) * invf
    fsb = jnp.broadcast_to(fs, (fn.shape[0], _K, _C))
    fcat = jnp.concatenate([fn, fsb], axis=-1)

    w = (fcat + pe) * pe
    agg = jnp.mean(w, axis=1) + jnp.max(w, axis=1)
    out_ref[...] = 0.5 * agg * (1.0 + jax.lax.erf(agg * np.float32(1.0 / np.sqrt(2.0))))


def _fused_tail(xk, xs, fk, fs, stats):
    grid = (_B * _S // _SE,)
    return pl.pallas_call(
        _tail_body,
        grid=grid,
        in_specs=[
            pl.BlockSpec((_SE, _K, _IN_DIM), lambda i: (i, 0, 0)),
            pl.BlockSpec((_SE, 1, _IN_DIM), lambda i: (i, 0, 0)),
            pl.BlockSpec((_SE, _K, _C), lambda i: (i, 0, 0)),
            pl.BlockSpec((_SE, 1, _C), lambda i: (i, 0, 0)),
            pl.BlockSpec((4, _K), lambda i: (0, 0)),
            pl.BlockSpec((1, 1, _OUT_DIM), lambda i: (0, 0, 0)),
        ],
        out_specs=pl.BlockSpec((_SE, _OUT_DIM), lambda i: (i, 0)),
        out_shape=jax.ShapeDtypeStruct((_B * _S, _OUT_DIM), jnp.float32),
    )(xk, xs, fk, fs, stats, jnp.asarray(_FVV_NP).reshape(1, 1, _OUT_DIM))


def _fps_body(xt_ref, out_ref):
    xt = xt_ref[...]
    x0, x1, x2 = xt[0], xt[1], xt[2]
    lane = jax.lax.broadcasted_iota(jnp.int32, (_B, _N), 1)

    def step(s, carry):
        dist, last, c0, c1, c2 = carry
        out_ref[pl.ds(s, 1), :] = last.reshape(1, _B)
        d0 = x0 - c0
        d1 = x1 - c1
        d2 = x2 - c2
        d = (d0 * d0 + d1 * d1) + d2 * d2
        dist = jnp.minimum(dist, d)
        m = jnp.max(dist, axis=1, keepdims=True)
        cand = jnp.where(dist == m, lane, _N)
        nxt = jnp.min(cand, axis=1).astype(jnp.int32)
        oh = lane == nxt[:, None]
        z = jnp.float32(0.0)
        n0 = jnp.sum(jnp.where(oh, x0, z), axis=1, keepdims=True)
        n1 = jnp.sum(jnp.where(oh, x1, z), axis=1, keepdims=True)
        n2 = jnp.sum(jnp.where(oh, x2, z), axis=1, keepdims=True)
        return dist, nxt, n0, n1, n2

    init = (jnp.full((_B, _N), 1e10, jnp.float32),
            jnp.zeros((_B,), jnp.int32),
            xt[0, :, 0:1], xt[1, :, 0:1], xt[2, :, 0:1])
    jax.lax.fori_loop(0, _S, step, init)


def _fps_pallas(xyz):
    xt = jnp.transpose(xyz, (2, 0, 1))
    out = pl.pallas_call(
        _fps_body,
        out_shape=jax.ShapeDtypeStruct((_S, _B), jnp.int32),
    )(xt)
    return out.T


def _knn_jax(xyz, new_xyz):
    d = -2.0 * jnp.matmul(new_xyz, jnp.swapaxes(xyz, 1, 2))
    d = d + jnp.sum(new_xyz ** 2, axis=-1)[:, :, None]
    d = d + jnp.sum(xyz ** 2, axis=-1)[:, None, :]
    _, gi = jax.lax.top_k(-d, _K)
    return gi


def kernel(xyz, feat):
    fps_idx = _fps_pallas(xyz)
    take = jax.vmap(lambda p, i: p[i])
    xyz_s = take(xyz, fps_idx)
    feat_s = take(feat, fps_idx)
    idx_knn = _knn_jax(xyz, xyz_s)
    xyz_k = take(xyz, idx_knn.reshape(_B, -1)).reshape(_B, _S, _K, _IN_DIM)
    feat_k = take(feat, idx_knn.reshape(_B, -1)).reshape(_B, _S, _K, _C)

    xk = xyz_k.reshape(_B * _S, _K, _IN_DIM)
    xs = xyz_s.reshape(_B * _S, 1, _IN_DIM)
    fk = feat_k.reshape(_B * _S, _K, _C)
    fs = feat_s.reshape(_B * _S, 1, _C)

    xmom, fmom = _compute_moments(xk, xs, fk, fs)
    stats = _finalize_stats(xmom, fmom)
    out = _fused_tail(xk, xs, fk, fs, stats)
    return out.reshape(_B, _S, _OUT_DIM)


# speedup vs baseline: 2.0755x; 1.3645x over previous
import functools
import math

import jax
import jax.numpy as jnp
import numpy as np
from jax import lax
from jax.experimental import pallas as pl
from jax.experimental.pallas import tpu as pltpu
from jax.experimental.pallas import tpu_sc as plsc

_B, _N, _C = 8, 4096, 64
_S, _K = 1024, 32
_IN_DIM = 3
_OUT_DIM = 128
_SIGMA = 0.26
_BASELINE = 0.1
_SCALING = 10.0
_EPS = 1e-06

_feat_dim = math.ceil(_OUT_DIM / _IN_DIM)
_feat_num = _feat_dim * _IN_DIM
_OUT_IDX_NP = np.linspace(0, _feat_num - 1, _OUT_DIM).astype(np.int64)
_FEAT_VAL_NP = np.linspace(-1.0, 1.0, _feat_dim + 2)[1:-1].astype(np.float32)
_COORD_OF_LANE = (_OUT_IDX_NP // _feat_dim).astype(np.int32)
_FVV_NP = _FEAT_VAL_NP[(_OUT_IDX_NP % _feat_dim)].astype(np.float32)
assert (_COORD_OF_LANE == np.where(np.arange(128) < 43, 0,
                                   np.where(np.arange(128) < 86, 1, 2))).all()

_NX = _B * _S * _IN_DIM
_NF = _B * _S * _C
_SK = _S * _K


_SD = 32


def _moments_body(xk_ref, xs_ref, fk_ref, fs_ref, xmom_ref, fmom_ref):
    b = pl.program_id(0)
    st = pl.program_id(1)
    cx = xk_ref[...] - xs_ref[...]
    cf = fk_ref[...] - fs_ref[...]
    xs0 = jnp.sum(cx, axis=0)
    xs1 = jnp.sum(cx * cx, axis=0)
    xm = jnp.stack([xs0, xs1], axis=0)[None]
    fs0 = jnp.sum(cf, axis=(0, 2))
    fs1 = jnp.sum(cf * cf, axis=(0, 2))
    fm = jnp.stack([fs0, fs1], axis=0)

    @pl.when(st == 0)
    def _():
        xmom_ref[...] = xm

    @pl.when(st != 0)
    def _():
        xmom_ref[...] += xm

    @pl.when(jnp.logical_and(b == 0, st == 0))
    def _():
        fmom_ref[...] = fm

    @pl.when(jnp.logical_not(jnp.logical_and(b == 0, st == 0)))
    def _():
        fmom_ref[...] += fm


def _compute_moments(xk, xs, fk, fs):
    grid = (_B, _S // _SD)
    return pl.pallas_call(
        _moments_body,
        grid=grid,
        in_specs=[
            pl.BlockSpec((_SD, _K, _IN_DIM), lambda b, st: (b * (_S // _SD) + st, 0, 0)),
            pl.BlockSpec((_SD, 1, _IN_DIM), lambda b, st: (b * (_S // _SD) + st, 0, 0)),
            pl.BlockSpec((_SD, _K, _C), lambda b, st: (b * (_S // _SD) + st, 0, 0)),
            pl.BlockSpec((_SD, 1, _C), lambda b, st: (b * (_S // _SD) + st, 0, 0)),
        ],
        out_specs=[
            pl.BlockSpec((1, 2, _K, _IN_DIM), lambda b, st: (b, 0, 0, 0)),
            pl.BlockSpec((2, _K), lambda b, st: (0, 0)),
        ],
        out_shape=[
            jax.ShapeDtypeStruct((_B, 2, _K, _IN_DIM), jnp.float32),
            jax.ShapeDtypeStruct((2, _K), jnp.float32),
        ],
    )(xk, xs, fk, fs)


def _finalize_body(xmom_ref, fmom_ref, out_ref):
    xm = xmom_ref[...]
    fm = fmom_ref[...]
    ts = jnp.sum(xm[:, 0], axis=(0, 2))
    tq = jnp.sum(xm[:, 1], axis=(0, 2))
    nx = jnp.float32(_NX)
    varx = (tq - ts * ts / nx) / (nx - 1.0)
    stdx = jnp.maximum(jnp.sqrt(jnp.maximum(varx, 0.0)), 1e-5)
    invx = 1.0 / stdx

    nf = jnp.float32(_NF)
    varf = (fm[1] - fm[0] * fm[0] / nf) / (nf - 1.0)
    stdf = jnp.maximum(jnp.sqrt(jnp.maximum(varf, 0.0)), 1e-5)
    invf = 1.0 / stdf

    sb = jnp.sum(xm[:, 0] * invx[None, :, None], axis=1)
    qb = jnp.sum(xm[:, 1] * (invx * invx)[None, :, None], axis=1)
    nsk = jnp.float32(_SK)
    var_bi = (qb - sb * sb / nsk) / (nsk - 1.0)
    std_bi = jnp.sqrt(jnp.maximum(var_bi, 0.0))
    gs = jnp.mean(std_bi)
    sigma = _SIGMA * (1.0 + gs)
    inv_sigma = 1.0 / (sigma + _EPS)
    blend = jax.nn.sigmoid((gs - _BASELINE) * _SCALING)

    ones = jnp.ones((_K,), jnp.float32)
    out_ref[...] = jnp.stack([invx, invf, inv_sigma * ones, blend * ones], axis=0)


def _finalize_stats(xmom, fmom):
    return pl.pallas_call(
        _finalize_body,
        out_shape=jax.ShapeDtypeStruct((4, _K), jnp.float32),
    )(xmom, fmom)


_SE = 8


def _tail_body(xk_ref, xs_ref, fk_ref, fs_ref, st_ref, fvv_ref, out_ref):
    st = st_ref[...]
    invx = st[0][None, :, None]
    invf = st[1][None, :, None]
    inv_sigma = st[2:3, 0:1].reshape(1, 1, 1)
    blend = st[3:4, 0:1].reshape(1, 1, 1)

    x = xk_ref[...]
    xn = (x - xs_ref[...]) * invx

    lane = jax.lax.broadcasted_iota(jnp.int32, (1, 1, _OUT_DIM), 2)
    x0 = xn[:, :, 0:1]
    x1 = xn[:, :, 1:2]
    x2 = xn[:, :, 2:3]
    sel = jnp.where(lane < 43, x0, jnp.where(lane < 86, x1, x2))
    t = (sel - fvv_ref[...]) * inv_sigma
    rbf = jnp.exp(-0.5 * (t * t))
    cosn = jnp.cos(t)
    pe = blend * rbf + (1.0 - blend) * cosn

    f = fk_ref[...]
    fs = fs_ref[...]
    fn = (f - fs) * invf
    fsb = jnp.broadcast_to(fs, (fn.shape[0], _K, _C))
    fcat = jnp.concatenate([fn, fsb], axis=-1)

    w = (fcat + pe) * pe
    agg = jnp.mean(w, axis=1) + jnp.max(w, axis=1)
    out_ref[...] = 0.5 * agg * (1.0 + jax.lax.erf(agg * np.float32(1.0 / np.sqrt(2.0))))


def _fused_tail(xk, xs, fk, fs, stats):
    grid = (_B * _S // _SE,)
    return pl.pallas_call(
        _tail_body,
        grid=grid,
        in_specs=[
            pl.BlockSpec((_SE, _K, _IN_DIM), lambda i: (i, 0, 0)),
            pl.BlockSpec((_SE, 1, _IN_DIM), lambda i: (i, 0, 0)),
            pl.BlockSpec((_SE, _K, _C), lambda i: (i, 0, 0)),
            pl.BlockSpec((_SE, 1, _C), lambda i: (i, 0, 0)),
            pl.BlockSpec((4, _K), lambda i: (0, 0)),
            pl.BlockSpec((1, 1, _OUT_DIM), lambda i: (0, 0, 0)),
        ],
        out_specs=pl.BlockSpec((_SE, _OUT_DIM), lambda i: (i, 0)),
        out_shape=jax.ShapeDtypeStruct((_B * _S, _OUT_DIM), jnp.float32),
    )(xk, xs, fk, fs, stats, jnp.asarray(_FVV_NP).reshape(1, 1, _OUT_DIM))


def _fps_body(xt_ref, out_ref):
    xt = xt_ref[...]
    x0, x1, x2 = xt[0], xt[1], xt[2]
    lane = jax.lax.broadcasted_iota(jnp.int32, (_B, _N), 1)

    def step(s, carry):
        dist, last, c0, c1, c2 = carry
        out_ref[pl.ds(s, 1), :] = last.reshape(1, _B)
        d0 = x0 - c0
        d1 = x1 - c1
        d2 = x2 - c2
        d = (d0 * d0 + d1 * d1) + d2 * d2
        dist = jnp.minimum(dist, d)
        m = jnp.max(dist, axis=1, keepdims=True)
        cand = jnp.where(dist == m, lane, _N)
        nxt = jnp.min(cand, axis=1).astype(jnp.int32)
        oh = lane == nxt[:, None]
        z = jnp.float32(0.0)
        n0 = jnp.sum(jnp.where(oh, x0, z), axis=1, keepdims=True)
        n1 = jnp.sum(jnp.where(oh, x1, z), axis=1, keepdims=True)
        n2 = jnp.sum(jnp.where(oh, x2, z), axis=1, keepdims=True)
        return dist, nxt, n0, n1, n2

    init = (jnp.full((_B, _N), 1e10, jnp.float32),
            jnp.zeros((_B,), jnp.int32),
            xt[0, :, 0:1], xt[1, :, 0:1], xt[2, :, 0:1])
    jax.lax.fori_loop(0, _S, step, init)


def _fps_pallas(xyz):
    xt = jnp.transpose(xyz, (2, 0, 1))
    out = pl.pallas_call(
        _fps_body,
        out_shape=jax.ShapeDtypeStruct((_S, _B), jnp.int32),
    )(xt)
    return out.T


_NTILES = 32
_QPT = _B * _S // _NTILES
_NV = _N // 16
_GB = 8


def _merge16(lo_k, lo_v, hi_k, hi_v, sk, sv):
    rs_k = lax.rev(sk, (0,))
    rs_v = lax.rev(sv, (0,))
    m1 = hi_k <= rs_k
    t_k = jnp.where(m1, hi_k, rs_k)
    t_v = jnp.where(m1, hi_v, rs_v)
    t_k, t_v = plsc.sort_key_val(t_k, t_v)
    r_k = lax.rev(t_k, (0,))
    r_v = lax.rev(t_v, (0,))
    m2 = lo_k <= r_k
    u0k = jnp.where(m2, lo_k, r_k)
    u0v = jnp.where(m2, lo_v, r_v)
    u1k = jnp.where(m2, r_k, lo_k)
    u1v = jnp.where(m2, r_v, lo_v)
    nlo_k, nlo_v = plsc.sort_key_val(u0k, u0v)
    nhi_k, nhi_v = plsc.sort_key_val(u1k, u1v)
    return nlo_k, nlo_v, nhi_k, nhi_v, nhi_k[15]


def _round_bf16(x):
    u = plsc.bitcast(x, jnp.int32)
    lsb = lax.shift_right_logical(u, 16) & 1
    u2 = (u + 0x7FFF + lsb) & jnp.int32(-65536)
    return plsc.bitcast(u2, jnp.float32)


def _knn_sc_body(xt_hbm, qt_hbm, out_hbm, xbuf, qbuf, obuf, bfbuf):
    wid = lax.axis_index("s") * 2 + lax.axis_index("c")
    b = wid // (_NTILES // _B)
    base = wid * _QPT
    pltpu.sync_copy(xt_hbm.at[b], xbuf)
    pltpu.sync_copy(qt_hbm.at[pl.ds(base, _QPT)], qbuf)
    iota16 = lax.iota(jnp.int32, 16)
    inf_k = jnp.full((16,), jnp.float32(np.inf))
    zero_v = jnp.zeros((16,), jnp.int32)

    def prep(v, _unused):
        col = v * 16
        px = xbuf[0, pl.ds(col, 16)]
        py = xbuf[1, pl.ds(col, 16)]
        pz = xbuf[2, pl.ds(col, 16)]
        bfbuf[0, pl.ds(col, 16)] = _round_bf16(px)
        bfbuf[1, pl.ds(col, 16)] = _round_bf16(py)
        bfbuf[2, pl.ds(col, 16)] = _round_bf16(pz)
        bfbuf[3, pl.ds(col, 16)] = (px * px + py * py) + pz * pz
        return _unused

    lax.fori_loop(0, _NV, prep, jnp.int32(0))

    def per_query(q, _unused):
        qrow = qbuf[q]
        qx = jnp.full((16,), qrow[0])
        qy = jnp.full((16,), qrow[1])
        qz = jnp.full((16,), qrow[2])
        s2q = (qx * qx + qy * qy) + qz * qz
        bqx = _round_bf16(qx)
        bqy = _round_bf16(qy)
        bqz = _round_bf16(qz)

        def dist(col):
            dot = (bqx * bfbuf[0, pl.ds(col, 16)]
                   + bqy * bfbuf[1, pl.ds(col, 16)]) \
                + bqz * bfbuf[2, pl.ds(col, 16)]
            return ((-2.0) * dot + s2q) + bfbuf[3, pl.ds(col, 16)]

        def batch_step(g, carry):
            col0 = g * (16 * _GB)
            ds = [dist(col0 + 16 * j) for j in range(_GB)]
            mall = ds[0]
            for j in range(1, _GB):
                mall = jnp.minimum(mall, ds[j])
            mk, _ = plsc.sort_key_val(mall, iota16)

            def do_merge(carry):
                for j in range(_GB):
                    sk, sv = plsc.sort_key_val(ds[j], iota16 + (col0 + 16 * j))

                    def yes(c, sk=sk, sv=sv):
                        lo_k, lo_v, hi_k, hi_v, _tau = c
                        return _merge16(lo_k, lo_v, hi_k, hi_v, sk, sv)

                    carry = lax.cond(sk[0] < carry[4], yes, lambda c: c, carry)
                return carry

            return lax.cond(mk[0] < carry[4], do_merge, lambda c: c, carry)

        init = (inf_k, zero_v, inf_k, zero_v, jnp.float32(np.inf))
        lo_k, lo_v, hi_k, hi_v, tau = lax.fori_loop(
            0, _NV // _GB, batch_step, init)
        obuf[2 * q] = lo_v
        obuf[2 * q + 1] = hi_v
        return _unused

    lax.fori_loop(0, _QPT, per_query, jnp.int32(0))
    pltpu.sync_copy(obuf, out_hbm.at[pl.ds(2 * base, 2 * _QPT)])


def _knn_sc(xyz, xyz_s):
    xt = jnp.transpose(xyz, (0, 2, 1))
    qt = jnp.zeros((_B * _S, 16), jnp.float32)
    qt = qt.at[:, :_IN_DIM].set(xyz_s.reshape(_B * _S, _IN_DIM))
    mesh = plsc.VectorSubcoreMesh(
        core_axis_name="c", subcore_axis_name="s", num_cores=2, num_subcores=16)
    knn = pl.kernel(
        _knn_sc_body,
        out_type=jax.ShapeDtypeStruct((2 * _B * _S, 16), jnp.int32),
        mesh=mesh,
        compiler_params=pltpu.CompilerParams(needs_layout_passes=False),
        scratch_types=[
            pltpu.VMEM((_IN_DIM, _N), jnp.float32),
            pltpu.VMEM((_QPT, 16), jnp.float32),
            pltpu.VMEM((2 * _QPT, 16), jnp.int32),
            pltpu.VMEM((4, _N), jnp.float32),
        ],
    )
    return knn(xt, qt).reshape(_B, _S, _K)


def _knn_jax(xyz, new_xyz):
    d = -2.0 * jnp.matmul(new_xyz, jnp.swapaxes(xyz, 1, 2))
    d = d + jnp.sum(new_xyz ** 2, axis=-1)[:, :, None]
    d = d + jnp.sum(xyz ** 2, axis=-1)[:, None, :]
    _, gi = jax.lax.top_k(-d, _K)
    return gi


def kernel(xyz, feat):
    fps_idx = _fps_pallas(xyz)
    take = jax.vmap(lambda p, i: p[i])
    xyz_s = take(xyz, fps_idx)
    feat_s = take(feat, fps_idx)
    idx_knn = _knn_sc(xyz, xyz_s)
    xyz_k = take(xyz, idx_knn.reshape(_B, -1)).reshape(_B, _S, _K, _IN_DIM)
    feat_k = take(feat, idx_knn.reshape(_B, -1)).reshape(_B, _S, _K, _C)

    xk = xyz_k.reshape(_B * _S, _K, _IN_DIM)
    xs = xyz_s.reshape(_B * _S, 1, _IN_DIM)
    fk = feat_k.reshape(_B * _S, _K, _C)
    fs = feat_s.reshape(_B * _S, 1, _C)

    xmom, fmom = _compute_moments(xk, xs, fk, fs)
    stats = _finalize_stats(xmom, fmom)
    out = _fused_tail(xk, xs, fk, fs, stats)
    return out.reshape(_B, _S, _OUT_DIM)


# speedup vs baseline: 6.4300x; 3.0980x over previous
import functools
import math

import jax
import jax.numpy as jnp
import numpy as np
from jax import lax
from jax.experimental import pallas as pl
from jax.experimental.pallas import tpu as pltpu
from jax.experimental.pallas import tpu_sc as plsc

_B, _N, _C = 8, 4096, 64
_S, _K = 1024, 32
_IN_DIM = 3
_OUT_DIM = 128
_SIGMA = 0.26
_BASELINE = 0.1
_SCALING = 10.0
_EPS = 1e-06

_feat_dim = math.ceil(_OUT_DIM / _IN_DIM)
_feat_num = _feat_dim * _IN_DIM
_OUT_IDX_NP = np.linspace(0, _feat_num - 1, _OUT_DIM).astype(np.int64)
_FEAT_VAL_NP = np.linspace(-1.0, 1.0, _feat_dim + 2)[1:-1].astype(np.float32)
_COORD_OF_LANE = (_OUT_IDX_NP // _feat_dim).astype(np.int32)
_FVV_NP = _FEAT_VAL_NP[(_OUT_IDX_NP % _feat_dim)].astype(np.float32)
assert (_COORD_OF_LANE == np.where(np.arange(128) < 43, 0,
                                   np.where(np.arange(128) < 86, 1, 2))).all()

_NX = _B * _S * _IN_DIM
_NF = _B * _S * _C
_SK = _S * _K


_SD = 32


def _moments_body(g_ref, xs_ref, fs_ref, xmom_ref, fmom_ref):
    b = pl.program_id(0)
    st = pl.program_id(1)
    g = g_ref[...]
    cx = g[:, :, _C:_C + _IN_DIM] - xs_ref[...]
    cf = g[:, :, :_C] - fs_ref[...]
    xs0 = jnp.sum(cx, axis=0)
    xs1 = jnp.sum(cx * cx, axis=0)
    xm = jnp.stack([xs0, xs1], axis=0)[None]
    fs0 = jnp.sum(cf, axis=(0, 2))
    fs1 = jnp.sum(cf * cf, axis=(0, 2))
    fm = jnp.stack([fs0, fs1], axis=0)

    @pl.when(st == 0)
    def _():
        xmom_ref[...] = xm

    @pl.when(st != 0)
    def _():
        xmom_ref[...] += xm

    @pl.when(jnp.logical_and(b == 0, st == 0))
    def _():
        fmom_ref[...] = fm

    @pl.when(jnp.logical_not(jnp.logical_and(b == 0, st == 0)))
    def _():
        fmom_ref[...] += fm


def _compute_moments(gk, xs, fs):
    grid = (_B, _S // _SD)
    return pl.pallas_call(
        _moments_body,
        grid=grid,
        in_specs=[
            pl.BlockSpec((_SD, _K, 128), lambda b, st: (b * (_S // _SD) + st, 0, 0)),
            pl.BlockSpec((_SD, 1, _IN_DIM), lambda b, st: (b * (_S // _SD) + st, 0, 0)),
            pl.BlockSpec((_SD, 1, _C), lambda b, st: (b * (_S // _SD) + st, 0, 0)),
        ],
        out_specs=[
            pl.BlockSpec((1, 2, _K, _IN_DIM), lambda b, st: (b, 0, 0, 0)),
            pl.BlockSpec((2, _K), lambda b, st: (0, 0)),
        ],
        out_shape=[
            jax.ShapeDtypeStruct((_B, 2, _K, _IN_DIM), jnp.float32),
            jax.ShapeDtypeStruct((2, _K), jnp.float32),
        ],
    )(gk, xs, fs)


def _finalize_body(xmom_ref, fmom_ref, out_ref):
    xm = xmom_ref[...]
    fm = fmom_ref[...]
    ts = jnp.sum(xm[:, 0], axis=(0, 2))
    tq = jnp.sum(xm[:, 1], axis=(0, 2))
    nx = jnp.float32(_NX)
    varx = (tq - ts * ts / nx) / (nx - 1.0)
    stdx = jnp.maximum(jnp.sqrt(jnp.maximum(varx, 0.0)), 1e-5)
    invx = 1.0 / stdx

    nf = jnp.float32(_NF)
    varf = (fm[1] - fm[0] * fm[0] / nf) / (nf - 1.0)
    stdf = jnp.maximum(jnp.sqrt(jnp.maximum(varf, 0.0)), 1e-5)
    invf = 1.0 / stdf

    sb = jnp.sum(xm[:, 0] * invx[None, :, None], axis=1)
    qb = jnp.sum(xm[:, 1] * (invx * invx)[None, :, None], axis=1)
    nsk = jnp.float32(_SK)
    var_bi = (qb - sb * sb / nsk) / (nsk - 1.0)
    std_bi = jnp.sqrt(jnp.maximum(var_bi, 0.0))
    gs = jnp.mean(std_bi)
    sigma = _SIGMA * (1.0 + gs)
    inv_sigma = 1.0 / (sigma + _EPS)
    blend = jax.nn.sigmoid((gs - _BASELINE) * _SCALING)

    ones = jnp.ones((_K,), jnp.float32)
    out_ref[...] = jnp.stack([invx, invf, inv_sigma * ones, blend * ones], axis=0)


def _finalize_stats(xmom, fmom):
    return pl.pallas_call(
        _finalize_body,
        out_shape=jax.ShapeDtypeStruct((4, _K), jnp.float32),
    )(xmom, fmom)


_SE = 8


def _tail_body(g_ref, xs_ref, fs_ref, st_ref, fvv_ref, out_ref):
    st = st_ref[...]
    invx = st[0][None, :, None]
    invf = st[1][None, :, None]
    inv_sigma = st[2:3, 0:1].reshape(1, 1, 1)
    blend = st[3:4, 0:1].reshape(1, 1, 1)

    g = g_ref[...]
    x = g[:, :, _C:_C + _IN_DIM]
    xn = (x - xs_ref[...]) * invx

    lane = jax.lax.broadcasted_iota(jnp.int32, (1, 1, _OUT_DIM), 2)
    x0 = xn[:, :, 0:1]
    x1 = xn[:, :, 1:2]
    x2 = xn[:, :, 2:3]
    sel = jnp.where(lane < 43, x0, jnp.where(lane < 86, x1, x2))
    t = (sel - fvv_ref[...]) * inv_sigma
    rbf = jnp.exp(-0.5 * (t * t))
    cosn = jnp.cos(t)
    pe = blend * rbf + (1.0 - blend) * cosn

    f = g[:, :, :_C]
    fs = fs_ref[...]
    fn = (f - fs) * invf
    fsb = jnp.broadcast_to(fs, (fn.shape[0], _K, _C))
    fcat = jnp.concatenate([fn, fsb], axis=-1)

    w = (fcat + pe) * pe
    agg = jnp.mean(w, axis=1) + jnp.max(w, axis=1)
    out_ref[...] = 0.5 * agg * (1.0 + jax.lax.erf(agg * np.float32(1.0 / np.sqrt(2.0))))


def _fused_tail(gk, xs, fs, stats):
    grid = (_B * _S // _SE,)
    return pl.pallas_call(
        _tail_body,
        grid=grid,
        in_specs=[
            pl.BlockSpec((_SE, _K, 128), lambda i: (i, 0, 0)),
            pl.BlockSpec((_SE, 1, _IN_DIM), lambda i: (i, 0, 0)),
            pl.BlockSpec((_SE, 1, _C), lambda i: (i, 0, 0)),
            pl.BlockSpec((4, _K), lambda i: (0, 0)),
            pl.BlockSpec((1, 1, _OUT_DIM), lambda i: (0, 0, 0)),
        ],
        out_specs=pl.BlockSpec((_SE, _OUT_DIM), lambda i: (i, 0)),
        out_shape=jax.ShapeDtypeStruct((_B * _S, _OUT_DIM), jnp.float32),
    )(gk, xs, fs, stats, jnp.asarray(_FVV_NP).reshape(1, 1, _OUT_DIM))


def _fps_body(xt_ref, out_ref):
    xt = xt_ref[...]
    x0, x1, x2 = xt[0], xt[1], xt[2]
    lane = jax.lax.broadcasted_iota(jnp.int32, (_B, _N), 1)

    def step(s, carry):
        dist, last, c0, c1, c2 = carry
        out_ref[pl.ds(s, 1), :] = last.reshape(1, _B)
        d0 = x0 - c0
        d1 = x1 - c1
        d2 = x2 - c2
        d = (d0 * d0 + d1 * d1) + d2 * d2
        dist = jnp.minimum(dist, d)
        m = jnp.max(dist, axis=1, keepdims=True)
        cand = jnp.where(dist == m, lane, _N)
        nxt = jnp.min(cand, axis=1).astype(jnp.int32)
        oh = lane == nxt[:, None]
        z = jnp.float32(0.0)
        n0 = jnp.sum(jnp.where(oh, x0, z), axis=1, keepdims=True)
        n1 = jnp.sum(jnp.where(oh, x1, z), axis=1, keepdims=True)
        n2 = jnp.sum(jnp.where(oh, x2, z), axis=1, keepdims=True)
        return dist, nxt, n0, n1, n2

    init = (jnp.full((_B, _N), 1e10, jnp.float32),
            jnp.zeros((_B,), jnp.int32),
            xt[0, :, 0:1], xt[1, :, 0:1], xt[2, :, 0:1])
    jax.lax.fori_loop(0, _S, step, init)


def _fps_pallas(xyz):
    xt = jnp.transpose(xyz, (2, 0, 1))
    out = pl.pallas_call(
        _fps_body,
        out_shape=jax.ShapeDtypeStruct((_S, _B), jnp.int32),
    )(xt)
    return out.T


_NTILES = 32
_QPT = _B * _S // _NTILES
_NV = _N // 16
_GB = 8


def _merge16(lo_k, lo_v, hi_k, hi_v, sk, sv):
    rs_k = lax.rev(sk, (0,))
    rs_v = lax.rev(sv, (0,))
    m1 = hi_k <= rs_k
    t_k = jnp.where(m1, hi_k, rs_k)
    t_v = jnp.where(m1, hi_v, rs_v)
    t_k, t_v = plsc.sort_key_val(t_k, t_v)
    r_k = lax.rev(t_k, (0,))
    r_v = lax.rev(t_v, (0,))
    m2 = lo_k <= r_k
    u0k = jnp.where(m2, lo_k, r_k)
    u0v = jnp.where(m2, lo_v, r_v)
    u1k = jnp.where(m2, r_k, lo_k)
    u1v = jnp.where(m2, r_v, lo_v)
    nlo_k, nlo_v = plsc.sort_key_val(u0k, u0v)
    nhi_k, nhi_v = plsc.sort_key_val(u1k, u1v)
    return nlo_k, nlo_v, nhi_k, nhi_v, nhi_k[15]


def _round_bf16(x):
    u = plsc.bitcast(x, jnp.int32)
    lsb = lax.shift_right_logical(u, 16) & 1
    u2 = (u + 0x7FFF + lsb) & jnp.int32(-65536)
    return plsc.bitcast(u2, jnp.float32)


_GCH = 512
_NCH = _QPT * _K // _GCH


def _knn_sc_body(xt_hbm, qt_hbm, tbl_hbm, gk_hbm,
                 xbuf, qbuf, obuf, bfbuf, idxg, gbuf, gsem):
    wid = lax.axis_index("s") * 2 + lax.axis_index("c")
    b = wid // (_NTILES // _B)
    base = wid * _QPT
    pltpu.sync_copy(xt_hbm.at[b], xbuf)
    pltpu.sync_copy(qt_hbm.at[pl.ds(wid * (_QPT // 8), _QPT // 8)], qbuf)
    iota16 = lax.iota(jnp.int32, 16)
    inf_k = jnp.full((16,), jnp.float32(np.inf))
    zero_v = jnp.zeros((16,), jnp.int32)

    def prep(v, _unused):
        col = v * 16
        px = xbuf[0, pl.ds(col, 16)]
        py = xbuf[1, pl.ds(col, 16)]
        pz = xbuf[2, pl.ds(col, 16)]
        bfbuf[3, pl.ds(col, 16)] = (px * px + py * py) + pz * pz
        bfbuf[0, pl.ds(col, 16)] = _round_bf16(px)
        bfbuf[1, pl.ds(col, 16)] = _round_bf16(py)
        bfbuf[2, pl.ds(col, 16)] = _round_bf16(pz)
        return _unused

    lax.fori_loop(0, _NV, prep, jnp.int32(0))

    def per_query(q, _unused):
        qrow = qbuf[q // 8, pl.ds((q % 8) * 16, 16)]
        qx = jnp.full((16,), qrow[0])
        qy = jnp.full((16,), qrow[1])
        qz = jnp.full((16,), qrow[2])
        s2q = (qx * qx + qy * qy) + qz * qz
        bqx = _round_bf16(qx)
        bqy = _round_bf16(qy)
        bqz = _round_bf16(qz)

        def dist(col):
            dot = (bqx * bfbuf[0, pl.ds(col, 16)]
                   + bqy * bfbuf[1, pl.ds(col, 16)]) \
                + bqz * bfbuf[2, pl.ds(col, 16)]
            return ((-2.0) * dot + s2q) + bfbuf[3, pl.ds(col, 16)]

        def batch_step(g, carry):
            col0 = g * (16 * _GB)
            ds = [dist(col0 + 16 * j) for j in range(_GB)]
            mall = ds[0]
            for j in range(1, _GB):
                mall = jnp.minimum(mall, ds[j])
            mk, _ = plsc.sort_key_val(mall, iota16)

            def do_merge(carry):
                for j in range(_GB):
                    sk, sv = plsc.sort_key_val(ds[j], iota16 + (col0 + 16 * j))

                    def yes(c, sk=sk, sv=sv):
                        lo_k, lo_v, hi_k, hi_v, _tau = c
                        return _merge16(lo_k, lo_v, hi_k, hi_v, sk, sv)

                    carry = lax.cond(sk[0] < carry[4], yes, lambda c: c, carry)
                return carry

            return lax.cond(mk[0] < carry[4], do_merge, lambda c: c, carry)

        init = (inf_k, zero_v, inf_k, zero_v, jnp.float32(np.inf))
        lo_k, lo_v, hi_k, hi_v, tau = lax.fori_loop(
            0, _NV // _GB, batch_step, init)
        obuf[q // 4, pl.ds((q % 4) * 32, 16)] = lo_v
        obuf[q // 4, pl.ds((q % 4) * 32 + 16, 16)] = hi_v
        return _unused

    lax.fori_loop(0, _QPT, per_query, jnp.int32(0))

    bN = b * _N

    rows_per_ch = _GCH // 16
    rbase = wid * (_QPT * _K)
    for ch in range(_NCH):
        ib = idxg[ch % 2]
        for w in range(rows_per_ch):
            r = ch * rows_per_ch + w
            ib[pl.ds(w * 16, 16)] = obuf[r // 8, pl.ds((r % 8) * 16, 16)] + bN
        pltpu.async_copy(tbl_hbm.at[ib], gbuf, gsem).wait()
        pltpu.sync_copy(gbuf, gk_hbm.at[pl.ds(rbase + ch * _GCH, _GCH)])


def _knn_sc(xyz, xyz_s, feat):
    xt = jnp.transpose(xyz, (0, 2, 1))
    qt = jnp.zeros((_B * _S, 16), jnp.float32)
    qt = qt.at[:, :_IN_DIM].set(xyz_s.reshape(_B * _S, _IN_DIM))
    qt = qt.reshape(_B * _S // 8, 128)
    tbl = jnp.concatenate(
        [feat, xyz, jnp.zeros((_B, _N, 128 - _C - _IN_DIM), jnp.float32)],
        axis=-1).reshape(_B * _N, 128)
    mesh = plsc.VectorSubcoreMesh(
        core_axis_name="c", subcore_axis_name="s", num_cores=2, num_subcores=16)
    knn = pl.kernel(
        _knn_sc_body,
        out_type=jax.ShapeDtypeStruct((_B * _S * _K, 128), jnp.float32),
        mesh=mesh,
        compiler_params=pltpu.CompilerParams(needs_layout_passes=False),
        scratch_types=[
            pltpu.VMEM((_IN_DIM, _N), jnp.float32),
            pltpu.VMEM((_QPT // 8, 128), jnp.float32),
            pltpu.VMEM((2 * _QPT // 8, 128), jnp.int32),
            pltpu.VMEM((4, _N), jnp.float32),
            [pltpu.VMEM((_GCH,), jnp.int32) for _ in range(2)],
            pltpu.VMEM((_GCH, 128), jnp.float32),
            pltpu.SemaphoreType.DMA,
        ],
    )
    gk = knn(xt, qt, tbl)
    return gk.reshape(_B * _S, _K, 128)


def _knn_jax(xyz, new_xyz):
    d = -2.0 * jnp.matmul(new_xyz, jnp.swapaxes(xyz, 1, 2))
    d = d + jnp.sum(new_xyz ** 2, axis=-1)[:, :, None]
    d = d + jnp.sum(xyz ** 2, axis=-1)[:, None, :]
    _, gi = jax.lax.top_k(-d, _K)
    return gi


def kernel(xyz, feat):
    fps_idx = _fps_pallas(xyz)
    take = jax.vmap(lambda p, i: p[i])
    xyz_s = take(xyz, fps_idx)
    feat_s = take(feat, fps_idx)
    gk = _knn_sc(xyz, xyz_s, feat)

    xs = xyz_s.reshape(_B * _S, 1, _IN_DIM)
    fs = feat_s.reshape(_B * _S, 1, _C)

    xmom, fmom = _compute_moments(gk, xs, fs)
    stats = _finalize_stats(xmom, fmom)
    out = _fused_tail(gk, xs, fs, stats)
    return out.reshape(_B, _S, _OUT_DIM)
